# Initial kernel scaffold; baseline (speedup 1.0000x reference)
import jax
import jax.numpy as jnp
from jax.experimental import pallas as pl


def kernel(hidden_states, router_w, w1, w3, w2, shared_w1, shared_w3, shared_w2):
    raise NotImplementedError("write your pallas kernel here")


# speedup vs baseline: 1.2437x; 1.2437x over previous
import functools

import jax
import jax.numpy as jnp
from jax import lax
from jax.experimental import pallas as pl
from jax.experimental.pallas import tpu as pltpu

T, D, F, E = 2048, 1024, 2048, 8
BT = 128
NB = 24
P = NB * BT
NBP = 32
TC = 16


def _route_kernel(logits_ref, slot_ref, w_ref, bexp_ref):
    c = pl.program_id(0)
    logits = logits_ref[...]
    lane = lax.broadcasted_iota(jnp.int32, (T, E), 1)
    m = jnp.max(logits, axis=1, keepdims=True)
    e_idx = jnp.min(jnp.where(logits == m, lane, E), axis=1, keepdims=True)
    onehot = (lane == e_idx).astype(jnp.float32)
    counts = jnp.sum(onehot, axis=0, keepdims=True)
    nblk = jnp.floor((counts + (BT - 1)) / BT)
    ii = lax.broadcasted_iota(jnp.int32, (E, E), 0)
    jj = lax.broadcasted_iota(jnp.int32, (E, E), 1)
    excl = (ii < jj).astype(jnp.float32)
    bstart = jnp.dot(nblk, excl, preferred_element_type=jnp.float32)
    rstart = bstart * BT

    row0 = c * BT
    tj = lax.broadcasted_iota(jnp.int32, (BT, T), 1)
    ti = row0 + lax.broadcasted_iota(jnp.int32, (BT, T), 0)
    tril = (tj < ti).astype(jnp.float32)
    csum = jnp.dot(tril, onehot, preferred_element_type=jnp.float32)
    logits_c = logits_ref[pl.ds(row0, BT), :]
    lane_c = lax.broadcasted_iota(jnp.int32, (BT, E), 1)
    m_c = jnp.max(logits_c, axis=1, keepdims=True)
    e_idx_c = jnp.min(jnp.where(logits_c == m_c, lane_c, E), axis=1, keepdims=True)
    oh_c = (lane_c == e_idx_c).astype(jnp.float32)
    rank = jnp.sum(csum * oh_c, axis=1, keepdims=True)
    rs_t = jnp.sum(oh_c * rstart, axis=1, keepdims=True)
    slot_ref[...] = (rs_t + rank).astype(jnp.int32)
    w_ref[...] = jax.nn.sigmoid(m_c)

    bi = lax.broadcasted_iota(jnp.int32, (NBP, E), 0).astype(jnp.float32)
    lane2 = lax.broadcasted_iota(jnp.int32, (NBP, E), 1)
    ind = (bi >= bstart) & (bi < bstart + nblk)
    bexp_ref[...] = jnp.sum(jnp.where(ind, lane2, 0), axis=1, keepdims=True)


def _moe_kernel(bexp_ref, xs_ref, ws_ref, w1_ref, w3_ref, w2_ref, y_ref):
    del bexp_ref
    x = xs_ref[...] * ws_ref[...]
    g = jnp.dot(x, w1_ref[0], preferred_element_type=jnp.float32)
    u = jnp.dot(x, w3_ref[0], preferred_element_type=jnp.float32)
    h = (g * jax.nn.sigmoid(g)) * u
    y_ref[...] = jnp.dot(h, w2_ref[0], preferred_element_type=jnp.float32)


def _shared_kernel(x_ref, w1_ref, w3_ref, w2_ref, y_ref):
    x = x_ref[...]
    g = jnp.dot(x, w1_ref[...], preferred_element_type=jnp.float32)
    u = jnp.dot(x, w3_ref[...], preferred_element_type=jnp.float32)
    h = (g * jax.nn.sigmoid(g)) * u
    y_ref[...] = jnp.dot(h, w2_ref[...], preferred_element_type=jnp.float32)


def kernel(hidden_states, router_w, w1, w3, w2, shared_w1, shared_w3, shared_w2):
    logits = hidden_states @ router_w

    slot2d, wtok, bexp2d = pl.pallas_call(
        _route_kernel,
        grid=(TC,),
        in_specs=[pl.BlockSpec((T, E), lambda c: (0, 0))],
        out_specs=[
            pl.BlockSpec((BT, 1), lambda c: (c, 0)),
            pl.BlockSpec((BT, 1), lambda c: (c, 0)),
            pl.BlockSpec((NBP, 1), lambda c: (0, 0)),
        ],
        out_shape=[
            jax.ShapeDtypeStruct((T, 1), jnp.int32),
            jax.ShapeDtypeStruct((T, 1), jnp.float32),
            jax.ShapeDtypeStruct((NBP, 1), jnp.int32),
        ],
    )(logits)
    slot = slot2d[:, 0]
    bexp = bexp2d[:NB, 0]

    tos = jnp.zeros((P,), jnp.int32).at[slot].set(jnp.arange(T, dtype=jnp.int32))
    w_sorted = jnp.zeros((P, 1), jnp.float32).at[slot].set(wtok)
    x_sorted = jnp.take(hidden_states, tos, axis=0)

    y_sorted = pl.pallas_call(
        _moe_kernel,
        grid_spec=pltpu.PrefetchScalarGridSpec(
            num_scalar_prefetch=1,
            grid=(NB,),
            in_specs=[
                pl.BlockSpec((BT, D), lambda i, bexp: (i, 0)),
                pl.BlockSpec((BT, 1), lambda i, bexp: (i, 0)),
                pl.BlockSpec((1, D, F), lambda i, bexp: (bexp[i], 0, 0)),
                pl.BlockSpec((1, D, F), lambda i, bexp: (bexp[i], 0, 0)),
                pl.BlockSpec((1, F, D), lambda i, bexp: (bexp[i], 0, 0)),
            ],
            out_specs=pl.BlockSpec((BT, D), lambda i, bexp: (i, 0)),
        ),
        out_shape=jax.ShapeDtypeStruct((P, D), jnp.float32),
    )(bexp, x_sorted, w_sorted, w1, w3, w2)

    BS = 256
    shared_out = pl.pallas_call(
        _shared_kernel,
        grid=(T // BS,),
        in_specs=[
            pl.BlockSpec((BS, D), lambda i: (i, 0)),
            pl.BlockSpec((D, F), lambda i: (0, 0)),
            pl.BlockSpec((D, F), lambda i: (0, 0)),
            pl.BlockSpec((F, D), lambda i: (0, 0)),
        ],
        out_specs=pl.BlockSpec((BS, D), lambda i: (i, 0)),
        out_shape=jax.ShapeDtypeStruct((T, D), jnp.float32),
    )(hidden_states, shared_w1, shared_w3, shared_w2)

    return shared_out + jnp.take(y_sorted, slot, axis=0)


# speedup vs baseline: 1.3356x; 1.0739x over previous
import functools

import jax
import jax.numpy as jnp
from jax import lax
from jax.experimental import pallas as pl
from jax.experimental.pallas import tpu as pltpu
from jax.experimental.pallas import tpu_sc as plsc

T, D, F, E = 2048, 1024, 2048, 8
BT = 128
NB = 24
P = NB * BT
NBP = 32
TC = 16


def _route_kernel(logits_ref, slot_ref, w_ref, bexp_ref):
    c = pl.program_id(0)
    logits = logits_ref[...]
    lane = lax.broadcasted_iota(jnp.int32, (T, E), 1)
    m = jnp.max(logits, axis=1, keepdims=True)
    e_idx = jnp.min(jnp.where(logits == m, lane, E), axis=1, keepdims=True)
    onehot = (lane == e_idx).astype(jnp.float32)
    counts = jnp.sum(onehot, axis=0, keepdims=True)
    nblk = jnp.floor((counts + (BT - 1)) / BT)
    ii = lax.broadcasted_iota(jnp.int32, (E, E), 0)
    jj = lax.broadcasted_iota(jnp.int32, (E, E), 1)
    excl = (ii < jj).astype(jnp.float32)
    bstart = jnp.dot(nblk, excl, preferred_element_type=jnp.float32)
    rstart = bstart * BT

    row0 = c * BT
    tj = lax.broadcasted_iota(jnp.int32, (BT, T), 1)
    ti = row0 + lax.broadcasted_iota(jnp.int32, (BT, T), 0)
    tril = (tj < ti).astype(jnp.float32)
    csum = jnp.dot(tril, onehot, preferred_element_type=jnp.float32)
    logits_c = logits_ref[pl.ds(row0, BT), :]
    lane_c = lax.broadcasted_iota(jnp.int32, (BT, E), 1)
    m_c = jnp.max(logits_c, axis=1, keepdims=True)
    e_idx_c = jnp.min(jnp.where(logits_c == m_c, lane_c, E), axis=1, keepdims=True)
    oh_c = (lane_c == e_idx_c).astype(jnp.float32)
    rank = jnp.sum(csum * oh_c, axis=1, keepdims=True)
    rs_t = jnp.sum(oh_c * rstart, axis=1, keepdims=True)
    slot_ref[...] = (rs_t + rank).astype(jnp.int32)
    w_ref[...] = jax.nn.sigmoid(m_c)

    bi = lax.broadcasted_iota(jnp.int32, (NBP, E), 0).astype(jnp.float32)
    lane2 = lax.broadcasted_iota(jnp.int32, (NBP, E), 1)
    ind = (bi >= bstart) & (bi < bstart + nblk)
    bexp_ref[...] = jnp.sum(jnp.where(ind, lane2, 0), axis=1, keepdims=True)


def _moe_kernel(bexp_ref, xs_ref, ws_ref, w1_ref, w3_ref, w2_ref, y_ref):
    del bexp_ref
    x = xs_ref[...] * ws_ref[...]
    g = jnp.dot(x, w1_ref[0], preferred_element_type=jnp.float32)
    u = jnp.dot(x, w3_ref[0], preferred_element_type=jnp.float32)
    h = (g * jax.nn.sigmoid(g)) * u
    y_ref[...] = jnp.dot(h, w2_ref[0], preferred_element_type=jnp.float32)


def _shared_kernel(x_ref, w1_ref, w3_ref, w2_ref, y_ref):
    x = x_ref[...]
    g = jnp.dot(x, w1_ref[...], preferred_element_type=jnp.float32)
    u = jnp.dot(x, w3_ref[...], preferred_element_type=jnp.float32)
    h = (g * jax.nn.sigmoid(g)) * u
    y_ref[...] = jnp.dot(h, w2_ref[...], preferred_element_type=jnp.float32)


_SC_MESH = plsc.VectorSubcoreMesh(core_axis_name="c", subcore_axis_name="s")
_SC_PARAMS = pltpu.CompilerParams(needs_layout_passes=False)
NW = 32
GW = P // NW
CW = T // NW
CH = 32


def _wid():
    return lax.axis_index("s") * 2 + lax.axis_index("c")


@functools.partial(
    pl.kernel,
    out_type=[
        jax.ShapeDtypeStruct((P,), jnp.int32),
        jax.ShapeDtypeStruct((P,), jnp.float32),
    ],
    mesh=_SC_MESH,
    scratch_types=[
        pltpu.VMEM((T,), jnp.int32),
        pltpu.VMEM((T,), jnp.float32),
        pltpu.VMEM((P,), jnp.int32),
        pltpu.VMEM((P,), jnp.float32),
    ],
    compiler_params=_SC_PARAMS,
)
def _sc_scatter(slot_hbm, w_hbm, tos_hbm, wsort_hbm, slot_v, w_v, tos_v, wsort_v):
    @pl.when(_wid() == 0)
    def _():
        pltpu.sync_copy(slot_hbm, slot_v)
        pltpu.sync_copy(w_hbm, w_v)

        @pl.loop(0, P // 16)
        def _(i):
            tos_v[pl.ds(i * 16, 16)] = jnp.zeros((16,), jnp.int32)
            wsort_v[pl.ds(i * 16, 16)] = jnp.zeros((16,), jnp.float32)

        @pl.loop(0, T // 16)
        def _(i):
            s = slot_v[pl.ds(i * 16, 16)]
            t = i * 16 + lax.iota(jnp.int32, 16)
            plsc.store_scatter(tos_v, [s], t)
            plsc.store_scatter(wsort_v, [s], w_v[pl.ds(i * 16, 16)])

        pltpu.sync_copy(tos_v, tos_hbm)
        pltpu.sync_copy(wsort_v, wsort_hbm)


@functools.partial(
    pl.kernel,
    out_type=jax.ShapeDtypeStruct((P, D), jnp.float32),
    mesh=_SC_MESH,
    scratch_types=[
        pltpu.VMEM((GW,), jnp.int32),
        pltpu.VMEM((GW, D), jnp.float32),
        pltpu.SemaphoreType.DMA,
    ],
)
def _sc_gather(x_hbm, tos_hbm, out_hbm, idx_v, rows_v, sem):
    base = _wid() * GW
    pltpu.sync_copy(tos_hbm.at[pl.ds(base, GW)], idx_v)
    pltpu.async_copy(x_hbm.at[idx_v], rows_v, sem).wait()
    pltpu.sync_copy(rows_v, out_hbm.at[pl.ds(base, GW)])


@functools.partial(
    pl.kernel,
    out_type=jax.ShapeDtypeStruct((T, D), jnp.float32),
    mesh=_SC_MESH,
    scratch_types=[
        pltpu.VMEM((CH,), jnp.int32),
        pltpu.VMEM((CH, D), jnp.float32),
        pltpu.VMEM((CH, D), jnp.float32),
        pltpu.SemaphoreType.DMA,
    ],
)
def _sc_combine(y_hbm, sh_hbm, slot_hbm, out_hbm, idx_v, rows_v, sh_v, sem):
    @pl.loop(0, CW // CH)
    def _(k):
        base = _wid() * CW + k * CH
        pltpu.sync_copy(slot_hbm.at[pl.ds(base, CH)], idx_v)
        pltpu.async_copy(y_hbm.at[idx_v], rows_v, sem).wait()
        pltpu.sync_copy(sh_hbm.at[pl.ds(base, CH)], sh_v)

        @pl.loop(0, CH)
        def _(r):
            @pl.loop(0, D // 16)
            def _(cc):
                rows_v[r, pl.ds(cc * 16, 16)] += sh_v[r, pl.ds(cc * 16, 16)]

        pltpu.sync_copy(rows_v, out_hbm.at[pl.ds(base, CH)])


def kernel(hidden_states, router_w, w1, w3, w2, shared_w1, shared_w3, shared_w2):
    logits = hidden_states @ router_w

    slot2d, wtok, bexp2d = pl.pallas_call(
        _route_kernel,
        grid=(TC,),
        in_specs=[pl.BlockSpec((T, E), lambda c: (0, 0))],
        out_specs=[
            pl.BlockSpec((BT, 1), lambda c: (c, 0)),
            pl.BlockSpec((BT, 1), lambda c: (c, 0)),
            pl.BlockSpec((NBP, 1), lambda c: (0, 0)),
        ],
        out_shape=[
            jax.ShapeDtypeStruct((T, 1), jnp.int32),
            jax.ShapeDtypeStruct((T, 1), jnp.float32),
            jax.ShapeDtypeStruct((NBP, 1), jnp.int32),
        ],
    )(logits)
    slot = slot2d[:, 0]
    bexp = bexp2d[:NB, 0]

    tos, w_sorted1d = _sc_scatter(slot, wtok[:, 0])
    w_sorted = w_sorted1d.reshape(P, 1)
    x_sorted = _sc_gather(hidden_states, tos)

    y_sorted = pl.pallas_call(
        _moe_kernel,
        grid_spec=pltpu.PrefetchScalarGridSpec(
            num_scalar_prefetch=1,
            grid=(NB,),
            in_specs=[
                pl.BlockSpec((BT, D), lambda i, bexp: (i, 0)),
                pl.BlockSpec((BT, 1), lambda i, bexp: (i, 0)),
                pl.BlockSpec((1, D, F), lambda i, bexp: (bexp[i], 0, 0)),
                pl.BlockSpec((1, D, F), lambda i, bexp: (bexp[i], 0, 0)),
                pl.BlockSpec((1, F, D), lambda i, bexp: (bexp[i], 0, 0)),
            ],
            out_specs=pl.BlockSpec((BT, D), lambda i, bexp: (i, 0)),
        ),
        out_shape=jax.ShapeDtypeStruct((P, D), jnp.float32),
    )(bexp, x_sorted, w_sorted, w1, w3, w2)

    BS = 256
    shared_out = pl.pallas_call(
        _shared_kernel,
        grid=(T // BS,),
        in_specs=[
            pl.BlockSpec((BS, D), lambda i: (i, 0)),
            pl.BlockSpec((D, F), lambda i: (0, 0)),
            pl.BlockSpec((D, F), lambda i: (0, 0)),
            pl.BlockSpec((F, D), lambda i: (0, 0)),
        ],
        out_specs=pl.BlockSpec((BS, D), lambda i: (i, 0)),
        out_shape=jax.ShapeDtypeStruct((T, D), jnp.float32),
    )(hidden_states, shared_w1, shared_w3, shared_w2)

    return _sc_combine(y_sorted, shared_out, slot)


# speedup vs baseline: 1.5219x; 1.1394x over previous
import functools

import jax
import jax.numpy as jnp
from jax import lax
from jax.experimental import pallas as pl
from jax.experimental.pallas import tpu as pltpu
from jax.experimental.pallas import tpu_sc as plsc

T, D, F, E = 2048, 1024, 2048, 8
BT = 128
NB = 24
P = NB * BT
NBP = 32
TC = 16


def _route_kernel(logits_ref, slot_ref, w_ref, bexp_ref):
    c = pl.program_id(0)
    logits = logits_ref[...]
    lane = lax.broadcasted_iota(jnp.int32, (T, E), 1)
    m = jnp.max(logits, axis=1, keepdims=True)
    e_idx = jnp.min(jnp.where(logits == m, lane, E), axis=1, keepdims=True)
    onehot = (lane == e_idx).astype(jnp.float32)
    counts = jnp.sum(onehot, axis=0, keepdims=True)
    nblk = jnp.floor((counts + (BT - 1)) / BT)
    ii = lax.broadcasted_iota(jnp.int32, (E, E), 0)
    jj = lax.broadcasted_iota(jnp.int32, (E, E), 1)
    excl = (ii < jj).astype(jnp.float32)
    bstart = jnp.dot(nblk, excl, preferred_element_type=jnp.float32)
    rstart = bstart * BT

    row0 = c * BT
    tj = lax.broadcasted_iota(jnp.int32, (BT, T), 1)
    ti = row0 + lax.broadcasted_iota(jnp.int32, (BT, T), 0)
    tril = (tj < ti).astype(jnp.float32)
    csum = jnp.dot(tril, onehot, preferred_element_type=jnp.float32)
    logits_c = logits_ref[pl.ds(row0, BT), :]
    lane_c = lax.broadcasted_iota(jnp.int32, (BT, E), 1)
    m_c = jnp.max(logits_c, axis=1, keepdims=True)
    e_idx_c = jnp.min(jnp.where(logits_c == m_c, lane_c, E), axis=1, keepdims=True)
    oh_c = (lane_c == e_idx_c).astype(jnp.float32)
    rank = jnp.sum(csum * oh_c, axis=1, keepdims=True)
    rs_t = jnp.sum(oh_c * rstart, axis=1, keepdims=True)
    slot_ref[...] = (rs_t + rank).astype(jnp.int32)
    w_ref[...] = jax.nn.sigmoid(m_c)

    bi = lax.broadcasted_iota(jnp.int32, (NBP, E), 0).astype(jnp.float32)
    lane2 = lax.broadcasted_iota(jnp.int32, (NBP, E), 1)
    ind = (bi >= bstart) & (bi < bstart + nblk)
    bexp_ref[...] = jnp.sum(jnp.where(ind, lane2, 0), axis=1, keepdims=True)


def _moe_kernel(bexp_ref, xs_ref, ws_ref, w1_ref, w3_ref, w2_ref, y_ref):
    del bexp_ref
    x = xs_ref[...] * ws_ref[...]
    g = jnp.dot(x, w1_ref[0], preferred_element_type=jnp.float32)
    u = jnp.dot(x, w3_ref[0], preferred_element_type=jnp.float32)
    h = (g * jax.nn.sigmoid(g)) * u
    y_ref[...] = jnp.dot(h, w2_ref[0], preferred_element_type=jnp.float32)


def _shared_kernel(x_ref, w1_ref, w3_ref, w2_ref, y_ref):
    x = x_ref[...]
    g = jnp.dot(x, w1_ref[...], preferred_element_type=jnp.float32)
    u = jnp.dot(x, w3_ref[...], preferred_element_type=jnp.float32)
    h = (g * jax.nn.sigmoid(g)) * u
    y_ref[...] = jnp.dot(h, w2_ref[...], preferred_element_type=jnp.float32)


_SC_MESH = plsc.VectorSubcoreMesh(core_axis_name="c", subcore_axis_name="s")
_SC_PARAMS = pltpu.CompilerParams(needs_layout_passes=False)
NW = 32
GW = P // NW
CW = T // NW
CH = 32


def _wid():
    return lax.axis_index("s") * 2 + lax.axis_index("c")


@functools.partial(
    pl.kernel,
    out_type=[
        jax.ShapeDtypeStruct((P,), jnp.int32),
        jax.ShapeDtypeStruct((P,), jnp.float32),
    ],
    mesh=_SC_MESH,
    scratch_types=[
        pltpu.VMEM((T,), jnp.int32),
        pltpu.VMEM((T,), jnp.float32),
        pltpu.VMEM((P,), jnp.int32),
        pltpu.VMEM((P,), jnp.float32),
    ],
    compiler_params=_SC_PARAMS,
)
def _sc_scatter(slot_hbm, w_hbm, tos_hbm, wsort_hbm, slot_v, w_v, tos_v, wsort_v):
    @pl.when(_wid() == 0)
    def _():
        pltpu.sync_copy(slot_hbm, slot_v)
        pltpu.sync_copy(w_hbm, w_v)

        @pl.loop(0, P // 16)
        def _(i):
            tos_v[pl.ds(i * 16, 16)] = (i * 16 + lax.iota(jnp.int32, 16)) & (T - 1)
            wsort_v[pl.ds(i * 16, 16)] = jnp.zeros((16,), jnp.float32)

        @pl.loop(0, T // 16)
        def _(i):
            s = slot_v[pl.ds(i * 16, 16)]
            t = i * 16 + lax.iota(jnp.int32, 16)
            plsc.store_scatter(tos_v, [s], t)
            plsc.store_scatter(wsort_v, [s], w_v[pl.ds(i * 16, 16)])

        pltpu.sync_copy(tos_v, tos_hbm)
        pltpu.sync_copy(wsort_v, wsort_hbm)


GCH = 32
GNC = GW // GCH


@functools.partial(
    pl.kernel,
    out_type=jax.ShapeDtypeStruct((P, D), jnp.float32),
    mesh=_SC_MESH,
    scratch_types=[
        pltpu.VMEM((GW,), jnp.int32),
        pltpu.VMEM((GCH, D), jnp.float32),
        pltpu.VMEM((GCH, D), jnp.float32),
        pltpu.SemaphoreType.DMA,
        pltpu.SemaphoreType.DMA,
    ],
)
def _sc_gather(x_hbm, tos_hbm, out_hbm, idx_v, b0, b1, semg, semo):
    base = _wid() * GW
    pltpu.sync_copy(tos_hbm.at[pl.ds(base, GW)], idx_v)
    buf = (b0, b1)

    def start(k):
        return pltpu.async_copy(x_hbm.at[idx_v.at[pl.ds(k * GCH, GCH)]], buf[k % 2], semg)

    g = [None] * GNC
    o = [None] * GNC
    g[0] = start(0)
    g[1] = start(1)
    for k in range(GNC):
        g[k].wait()
        o[k] = pltpu.async_copy(buf[k % 2], out_hbm.at[pl.ds(base + k * GCH, GCH)], semo)
        if k + 2 < GNC:
            o[k].wait()
            g[k + 2] = start(k + 2)
    for k in range(max(0, GNC - 2), GNC):
        o[k].wait()


ECH = 16
ENC = CW // ECH


@functools.partial(
    pl.kernel,
    out_type=jax.ShapeDtypeStruct((T, D), jnp.float32),
    mesh=_SC_MESH,
    scratch_types=[
        pltpu.VMEM((ECH,), jnp.int32),
        pltpu.VMEM((ECH,), jnp.int32),
        pltpu.VMEM((ECH, D), jnp.float32),
        pltpu.VMEM((ECH, D), jnp.float32),
        pltpu.VMEM((ECH, D), jnp.float32),
        pltpu.VMEM((ECH, D), jnp.float32),
        pltpu.SemaphoreType.DMA,
        pltpu.SemaphoreType.DMA,
        pltpu.SemaphoreType.DMA,
    ],
)
def _sc_combine(y_hbm, sh_hbm, slot_hbm, out_hbm,
                idx0, idx1, r0, r1, s0, s1, semy, semsh, semo):
    base = _wid() * CW
    idx = (idx0, idx1)
    rbuf = (r0, r1)
    sbuf = (s0, s1)

    def start(k, b):
        lo = base + k * ECH
        pltpu.sync_copy(slot_hbm.at[pl.ds(lo, ECH)], idx[b])
        hy = pltpu.async_copy(y_hbm.at[idx[b]], rbuf[b], semy)
        hs = pltpu.async_copy(sh_hbm.at[pl.ds(lo, ECH)], sbuf[b], semsh)
        return hy, hs

    hands = [None] * ENC
    outh = [None] * ENC
    hands[0] = start(0, 0)
    hands[1] = start(1, 1)
    for k in range(ENC):
        b = k % 2
        hy, hs = hands[k]
        hy.wait()
        hs.wait()
        for r in range(ECH):
            @pl.loop(0, D // 16)
            def _(cc, r=r, b=b):
                rbuf[b][r, pl.ds(cc * 16, 16)] += sbuf[b][r, pl.ds(cc * 16, 16)]
        outh[k] = pltpu.async_copy(rbuf[b], out_hbm.at[pl.ds(base + k * ECH, ECH)], semo)
        if k + 2 < ENC:
            outh[k].wait()
            hands[k + 2] = start(k + 2, b)
    outh[ENC - 2].wait()
    outh[ENC - 1].wait()


def kernel(hidden_states, router_w, w1, w3, w2, shared_w1, shared_w3, shared_w2):
    BS = 256
    shared_out = pl.pallas_call(
        _shared_kernel,
        grid=(T // BS,),
        in_specs=[
            pl.BlockSpec((BS, D), lambda i: (i, 0)),
            pl.BlockSpec((D, F), lambda i: (0, 0)),
            pl.BlockSpec((D, F), lambda i: (0, 0)),
            pl.BlockSpec((F, D), lambda i: (0, 0)),
        ],
        out_specs=pl.BlockSpec((BS, D), lambda i: (i, 0)),
        out_shape=jax.ShapeDtypeStruct((T, D), jnp.float32),
    )(hidden_states, shared_w1, shared_w3, shared_w2)

    logits = hidden_states @ router_w

    slot2d, wtok, bexp2d = pl.pallas_call(
        _route_kernel,
        grid=(TC,),
        in_specs=[pl.BlockSpec((T, E), lambda c: (0, 0))],
        out_specs=[
            pl.BlockSpec((BT, 1), lambda c: (c, 0)),
            pl.BlockSpec((BT, 1), lambda c: (c, 0)),
            pl.BlockSpec((NBP, 1), lambda c: (0, 0)),
        ],
        out_shape=[
            jax.ShapeDtypeStruct((T, 1), jnp.int32),
            jax.ShapeDtypeStruct((T, 1), jnp.float32),
            jax.ShapeDtypeStruct((NBP, 1), jnp.int32),
        ],
    )(logits)
    slot = slot2d[:, 0]
    bexp = bexp2d[:NB, 0]

    tos, w_sorted1d = _sc_scatter(slot, wtok[:, 0])
    w_sorted = w_sorted1d.reshape(P, 1)
    x_sorted = _sc_gather(hidden_states, tos)

    y_sorted = pl.pallas_call(
        _moe_kernel,
        grid_spec=pltpu.PrefetchScalarGridSpec(
            num_scalar_prefetch=1,
            grid=(NB,),
            in_specs=[
                pl.BlockSpec((BT, D), lambda i, bexp: (i, 0)),
                pl.BlockSpec((BT, 1), lambda i, bexp: (i, 0)),
                pl.BlockSpec((1, D, F), lambda i, bexp: (bexp[i], 0, 0)),
                pl.BlockSpec((1, D, F), lambda i, bexp: (bexp[i], 0, 0)),
                pl.BlockSpec((1, F, D), lambda i, bexp: (bexp[i], 0, 0)),
            ],
            out_specs=pl.BlockSpec((BT, D), lambda i, bexp: (i, 0)),
        ),
        out_shape=jax.ShapeDtypeStruct((P, D), jnp.float32),
    )(bexp, x_sorted, w_sorted, w1, w3, w2)

    return _sc_combine(y_sorted, shared_out, slot)


# speedup vs baseline: 1.7203x; 1.1304x over previous
import functools

import jax
import jax.numpy as jnp
from jax import lax
from jax.experimental import pallas as pl
from jax.experimental.pallas import tpu as pltpu
from jax.experimental.pallas import tpu_sc as plsc

T, D, F, E = 2048, 1024, 2048, 8
BT = 256
NB = 16
P = NB * BT
RC = 128
TC = T // RC
NBP = 32


def _route_kernel(logits_ref, slot_ref, w_ref, bexp_ref, valid_ref):
    c = pl.program_id(0)
    logits = logits_ref[...]
    lane = lax.broadcasted_iota(jnp.int32, (T, E), 1)
    m = jnp.max(logits, axis=1, keepdims=True)
    e_idx = jnp.min(jnp.where(logits == m, lane, E), axis=1, keepdims=True)
    onehot = (lane == e_idx).astype(jnp.float32)
    counts = jnp.sum(onehot, axis=0, keepdims=True)
    nblk = jnp.floor((counts + (BT - 1)) / BT)
    ii = lax.broadcasted_iota(jnp.int32, (E, E), 0)
    jj = lax.broadcasted_iota(jnp.int32, (E, E), 1)
    excl = (ii < jj).astype(jnp.float32)
    bstart = jnp.dot(nblk, excl, preferred_element_type=jnp.float32)
    rstart = bstart * BT

    row0 = c * RC
    tj = lax.broadcasted_iota(jnp.int32, (RC, T), 1)
    ti = row0 + lax.broadcasted_iota(jnp.int32, (RC, T), 0)
    tril = (tj < ti).astype(jnp.float32)
    csum = jnp.dot(tril, onehot, preferred_element_type=jnp.float32)
    logits_c = logits_ref[pl.ds(row0, RC), :]
    lane_c = lax.broadcasted_iota(jnp.int32, (RC, E), 1)
    m_c = jnp.max(logits_c, axis=1, keepdims=True)
    e_idx_c = jnp.min(jnp.where(logits_c == m_c, lane_c, E), axis=1, keepdims=True)
    oh_c = (lane_c == e_idx_c).astype(jnp.float32)
    rank = jnp.sum(csum * oh_c, axis=1, keepdims=True)
    rs_t = jnp.sum(oh_c * rstart, axis=1, keepdims=True)
    slot_ref[...] = (rs_t + rank).astype(jnp.int32)
    w_ref[...] = jax.nn.sigmoid(m_c)

    bi = lax.broadcasted_iota(jnp.int32, (NBP, E), 0).astype(jnp.float32)
    lane2 = lax.broadcasted_iota(jnp.int32, (NBP, E), 1)
    ind = (bi >= bstart) & (bi < bstart + nblk)
    lastexp = jnp.max(jnp.where(counts > 0.0, lane2[:1], 0), axis=1, keepdims=True)
    used = jnp.sum(ind.astype(jnp.int32), axis=1, keepdims=True)
    bexp = jnp.sum(jnp.where(ind, lane2, 0), axis=1, keepdims=True)
    bexp_ref[...] = jnp.where(used > 0, bexp, lastexp)
    valid_ref[...] = used


def _moe_kernel(bexp_ref, valid_ref, xs_ref, ws_ref, w1_ref, w3_ref, w2_ref, y_ref):
    del bexp_ref
    i = pl.program_id(0)

    @pl.when(valid_ref[i] > 0)
    def _():
        x = xs_ref[...] * ws_ref[...]
        g = jnp.dot(x, w1_ref[0], preferred_element_type=jnp.float32)
        u = jnp.dot(x, w3_ref[0], preferred_element_type=jnp.float32)
        h = (g * jax.nn.sigmoid(g)) * u
        y_ref[...] = jnp.dot(h, w2_ref[0], preferred_element_type=jnp.float32)


def _shared_kernel(x_ref, routed_ref, w1_ref, w3_ref, w2_ref, y_ref):
    x = x_ref[...]
    g = jnp.dot(x, w1_ref[...], preferred_element_type=jnp.float32)
    u = jnp.dot(x, w3_ref[...], preferred_element_type=jnp.float32)
    h = (g * jax.nn.sigmoid(g)) * u
    y_ref[...] = routed_ref[...] + jnp.dot(h, w2_ref[...], preferred_element_type=jnp.float32)


_SC_MESH = plsc.VectorSubcoreMesh(core_axis_name="c", subcore_axis_name="s")
_SC_PARAMS = pltpu.CompilerParams(needs_layout_passes=False)
NW = 32
GCH = 32


def _wid():
    return lax.axis_index("s") * 2 + lax.axis_index("c")


@functools.partial(
    pl.kernel,
    out_type=[
        jax.ShapeDtypeStruct((P,), jnp.int32),
        jax.ShapeDtypeStruct((P,), jnp.float32),
    ],
    mesh=_SC_MESH,
    scratch_types=[
        pltpu.VMEM((T,), jnp.int32),
        pltpu.VMEM((T,), jnp.float32),
        pltpu.VMEM((P,), jnp.int32),
        pltpu.VMEM((P,), jnp.float32),
    ],
    compiler_params=_SC_PARAMS,
)
def _sc_scatter(slot_hbm, w_hbm, tos_hbm, wsort_hbm, slot_v, w_v, tos_v, wsort_v):
    @pl.when(_wid() == 0)
    def _():
        pltpu.sync_copy(slot_hbm, slot_v)
        pltpu.sync_copy(w_hbm, w_v)

        @pl.loop(0, P // 16)
        def _(i):
            tos_v[pl.ds(i * 16, 16)] = (i * 16 + lax.iota(jnp.int32, 16)) & (T - 1)
            wsort_v[pl.ds(i * 16, 16)] = jnp.zeros((16,), jnp.float32)

        @pl.loop(0, T // 16)
        def _(i):
            s = slot_v[pl.ds(i * 16, 16)]
            t = i * 16 + lax.iota(jnp.int32, 16)
            plsc.store_scatter(tos_v, [s], t)
            plsc.store_scatter(wsort_v, [s], w_v[pl.ds(i * 16, 16)])

        pltpu.sync_copy(tos_v, tos_hbm)
        pltpu.sync_copy(wsort_v, wsort_hbm)


def _make_row_gather(nrows):
    gw = nrows // NW
    gnc = gw // GCH

    @functools.partial(
        pl.kernel,
        out_type=jax.ShapeDtypeStruct((nrows, D), jnp.float32),
        mesh=_SC_MESH,
        scratch_types=[
            pltpu.VMEM((gw,), jnp.int32),
            pltpu.VMEM((GCH, D), jnp.float32),
            pltpu.VMEM((GCH, D), jnp.float32),
            pltpu.SemaphoreType.DMA,
            pltpu.SemaphoreType.DMA,
        ],
    )
    def gather(src_hbm, idx_hbm, out_hbm, idx_v, b0, b1, semg, semo):
        base = _wid() * gw
        pltpu.sync_copy(idx_hbm.at[pl.ds(base, gw)], idx_v)
        buf = (b0, b1)

        def start(k):
            return pltpu.async_copy(
                src_hbm.at[idx_v.at[pl.ds(k * GCH, GCH)]], buf[k % 2], semg)

        g = [None] * gnc
        o = [None] * gnc
        g[0] = start(0)
        if gnc > 1:
            g[1] = start(1)
        for k in range(gnc):
            g[k].wait()
            o[k] = pltpu.async_copy(
                buf[k % 2], out_hbm.at[pl.ds(base + k * GCH, GCH)], semo)
            if k + 2 < gnc:
                o[k].wait()
                g[k + 2] = start(k + 2)
        for k in range(max(0, gnc - 2), gnc):
            o[k].wait()

    return gather


_sc_gather_x = _make_row_gather(P)
_sc_gather_y = _make_row_gather(T)


def kernel(hidden_states, router_w, w1, w3, w2, shared_w1, shared_w3, shared_w2):
    logits = hidden_states @ router_w

    slot2d, wtok, bexp2d, valid2d = pl.pallas_call(
        _route_kernel,
        grid=(TC,),
        in_specs=[pl.BlockSpec((T, E), lambda c: (0, 0))],
        out_specs=[
            pl.BlockSpec((RC, 1), lambda c: (c, 0)),
            pl.BlockSpec((RC, 1), lambda c: (c, 0)),
            pl.BlockSpec((NBP, 1), lambda c: (0, 0)),
            pl.BlockSpec((NBP, 1), lambda c: (0, 0)),
        ],
        out_shape=[
            jax.ShapeDtypeStruct((T, 1), jnp.int32),
            jax.ShapeDtypeStruct((T, 1), jnp.float32),
            jax.ShapeDtypeStruct((NBP, 1), jnp.int32),
            jax.ShapeDtypeStruct((NBP, 1), jnp.int32),
        ],
    )(logits)
    slot = slot2d[:, 0]
    bexp = bexp2d[:NB, 0]
    valid = valid2d[:NB, 0]

    tos, w_sorted1d = _sc_scatter(slot, wtok[:, 0])
    w_sorted = w_sorted1d.reshape(P, 1)
    x_sorted = _sc_gather_x(hidden_states, tos)

    y_sorted = pl.pallas_call(
        _moe_kernel,
        grid_spec=pltpu.PrefetchScalarGridSpec(
            num_scalar_prefetch=2,
            grid=(NB,),
            in_specs=[
                pl.BlockSpec((BT, D), lambda i, bexp, valid: (i, 0)),
                pl.BlockSpec((BT, 1), lambda i, bexp, valid: (i, 0)),
                pl.BlockSpec((1, D, F), lambda i, bexp, valid: (bexp[i], 0, 0)),
                pl.BlockSpec((1, D, F), lambda i, bexp, valid: (bexp[i], 0, 0)),
                pl.BlockSpec((1, F, D), lambda i, bexp, valid: (bexp[i], 0, 0)),
            ],
            out_specs=pl.BlockSpec((BT, D), lambda i, bexp, valid: (i, 0)),
        ),
        out_shape=jax.ShapeDtypeStruct((P, D), jnp.float32),
    )(bexp, valid, x_sorted, w_sorted, w1, w3, w2)

    routed = _sc_gather_y(y_sorted, slot)

    BS = 256
    return pl.pallas_call(
        _shared_kernel,
        grid=(T // BS,),
        in_specs=[
            pl.BlockSpec((BS, D), lambda i: (i, 0)),
            pl.BlockSpec((BS, D), lambda i: (i, 0)),
            pl.BlockSpec((D, F), lambda i: (0, 0)),
            pl.BlockSpec((D, F), lambda i: (0, 0)),
            pl.BlockSpec((F, D), lambda i: (0, 0)),
        ],
        out_specs=pl.BlockSpec((BS, D), lambda i: (i, 0)),
        out_shape=jax.ShapeDtypeStruct((T, D), jnp.float32),
    )(hidden_states, routed, shared_w1, shared_w3, shared_w2)


# speedup vs baseline: 1.8749x; 1.0899x over previous
import functools

import jax
import jax.numpy as jnp
from jax import lax
from jax.experimental import pallas as pl
from jax.experimental.pallas import tpu as pltpu
from jax.experimental.pallas import tpu_sc as plsc

T, D, F, E = 2048, 1024, 2048, 8
BT = 256
NB = 16
P = NB * BT
RC = 128
TC = T // RC
NBP = 32


def _route_kernel(logits_ref, slot_ref, w_ref, bexp_ref, valid_ref):
    logits = logits_ref[...]
    lane = lax.broadcasted_iota(jnp.int32, (T, E), 1)
    m = jnp.max(logits, axis=1, keepdims=True)
    e_idx = jnp.min(jnp.where(logits == m, lane, E), axis=1, keepdims=True)
    onehot = (lane == e_idx).astype(jnp.float32)
    counts = jnp.sum(onehot, axis=0, keepdims=True)
    nblk = jnp.floor((counts + (BT - 1)) / BT)
    ii = lax.broadcasted_iota(jnp.int32, (E, E), 0)
    jj = lax.broadcasted_iota(jnp.int32, (E, E), 1)
    excl = (ii < jj).astype(jnp.float32)
    bstart = jnp.dot(nblk, excl, preferred_element_type=jnp.float32)
    rstart = bstart * BT

    ti = lax.broadcasted_iota(jnp.int32, (T, T), 0)
    tj = lax.broadcasted_iota(jnp.int32, (T, T), 1)
    tril = (tj < ti).astype(jnp.float32)
    csum = jnp.dot(tril, onehot, preferred_element_type=jnp.float32)
    rank = jnp.sum(csum * onehot, axis=1, keepdims=True)
    rs_t = jnp.sum(onehot * rstart, axis=1, keepdims=True)
    slot_ref[...] = (rs_t + rank).astype(jnp.int32)
    w_ref[...] = jax.nn.sigmoid(m)

    bi = lax.broadcasted_iota(jnp.int32, (NBP, E), 0).astype(jnp.float32)
    lane2 = lax.broadcasted_iota(jnp.int32, (NBP, E), 1)
    ind = (bi >= bstart) & (bi < bstart + nblk)
    lastexp = jnp.max(jnp.where(counts > 0.0, lane2[:1], 0), axis=1, keepdims=True)
    used = jnp.sum(ind.astype(jnp.int32), axis=1, keepdims=True)
    bexp = jnp.sum(jnp.where(ind, lane2, 0), axis=1, keepdims=True)
    bexp_ref[...] = jnp.where(used > 0, bexp, lastexp)
    valid_ref[...] = used


def _moe_kernel(bexp_ref, valid_ref, xs_ref, ws_ref, w1_ref, w3_ref, w2_ref, y_ref):
    del bexp_ref
    i = pl.program_id(0)

    @pl.when(valid_ref[i] > 0)
    def _():
        x = xs_ref[...] * ws_ref[...]
        g = jnp.dot(x, w1_ref[0], preferred_element_type=jnp.float32)
        u = jnp.dot(x, w3_ref[0], preferred_element_type=jnp.float32)
        h = (g * jax.nn.sigmoid(g)) * u
        y_ref[...] = jnp.dot(h, w2_ref[0], preferred_element_type=jnp.float32)


def _shared_kernel(x_ref, routed_ref, w1_ref, w3_ref, w2_ref, y_ref):
    x = x_ref[...].astype(jnp.bfloat16)
    w1b = w1_ref[...].astype(jnp.bfloat16)
    w3b = w3_ref[...].astype(jnp.bfloat16)
    g = jnp.dot(x, w1b, preferred_element_type=jnp.float32)
    u = jnp.dot(x, w3b, preferred_element_type=jnp.float32)
    h = ((g * jax.nn.sigmoid(g)) * u).astype(jnp.bfloat16)
    w2b = w2_ref[...].astype(jnp.bfloat16)
    y_ref[...] = routed_ref[...] + jnp.dot(h, w2b, preferred_element_type=jnp.float32)


_SC_MESH = plsc.VectorSubcoreMesh(core_axis_name="c", subcore_axis_name="s")
_SC_PARAMS = pltpu.CompilerParams(needs_layout_passes=False)
NW = 32
GCH = 32


def _wid():
    return lax.axis_index("s") * 2 + lax.axis_index("c")


@functools.partial(
    pl.kernel,
    out_type=[
        jax.ShapeDtypeStruct((P,), jnp.int32),
        jax.ShapeDtypeStruct((P,), jnp.float32),
    ],
    mesh=_SC_MESH,
    scratch_types=[
        pltpu.VMEM((T,), jnp.int32),
        pltpu.VMEM((T,), jnp.float32),
        pltpu.VMEM((P,), jnp.int32),
        pltpu.VMEM((P,), jnp.float32),
    ],
    compiler_params=_SC_PARAMS,
)
def _sc_scatter(slot_hbm, w_hbm, tos_hbm, wsort_hbm, slot_v, w_v, tos_v, wsort_v):
    @pl.when(_wid() == 0)
    def _():
        pltpu.sync_copy(slot_hbm, slot_v)
        pltpu.sync_copy(w_hbm, w_v)

        @pl.loop(0, P // 16)
        def _(i):
            tos_v[pl.ds(i * 16, 16)] = (i * 16 + lax.iota(jnp.int32, 16)) & (T - 1)
            wsort_v[pl.ds(i * 16, 16)] = jnp.zeros((16,), jnp.float32)

        @pl.loop(0, T // 16)
        def _(i):
            s = slot_v[pl.ds(i * 16, 16)]
            t = i * 16 + lax.iota(jnp.int32, 16)
            plsc.store_scatter(tos_v, [s], t)
            plsc.store_scatter(wsort_v, [s], w_v[pl.ds(i * 16, 16)])

        pltpu.sync_copy(tos_v, tos_hbm)
        pltpu.sync_copy(wsort_v, wsort_hbm)


def _make_row_gather(nrows):
    gw = nrows // NW
    gnc = gw // GCH

    @functools.partial(
        pl.kernel,
        out_type=jax.ShapeDtypeStruct((nrows, D), jnp.float32),
        mesh=_SC_MESH,
        scratch_types=[
            pltpu.VMEM((gw,), jnp.int32),
            pltpu.VMEM((GCH, D), jnp.float32),
            pltpu.VMEM((GCH, D), jnp.float32),
            pltpu.SemaphoreType.DMA,
            pltpu.SemaphoreType.DMA,
        ],
    )
    def gather(src_hbm, idx_hbm, out_hbm, idx_v, b0, b1, semg, semo):
        base = _wid() * gw
        pltpu.sync_copy(idx_hbm.at[pl.ds(base, gw)], idx_v)
        buf = (b0, b1)

        def start(k):
            return pltpu.async_copy(
                src_hbm.at[idx_v.at[pl.ds(k * GCH, GCH)]], buf[k % 2], semg)

        g = [None] * gnc
        o = [None] * gnc
        g[0] = start(0)
        if gnc > 1:
            g[1] = start(1)
        for k in range(gnc):
            g[k].wait()
            o[k] = pltpu.async_copy(
                buf[k % 2], out_hbm.at[pl.ds(base + k * GCH, GCH)], semo)
            if k + 2 < gnc:
                o[k].wait()
                g[k + 2] = start(k + 2)
        for k in range(max(0, gnc - 2), gnc):
            o[k].wait()

    return gather


_sc_gather_x = _make_row_gather(P)
_sc_gather_y = _make_row_gather(T)


def kernel(hidden_states, router_w, w1, w3, w2, shared_w1, shared_w3, shared_w2):
    logits = hidden_states @ router_w

    slot2d, wtok, bexp2d, valid2d = pl.pallas_call(
        _route_kernel,
        out_shape=[
            jax.ShapeDtypeStruct((T, 1), jnp.int32),
            jax.ShapeDtypeStruct((T, 1), jnp.float32),
            jax.ShapeDtypeStruct((NBP, 1), jnp.int32),
            jax.ShapeDtypeStruct((NBP, 1), jnp.int32),
        ],
    )(logits)
    slot = slot2d[:, 0]
    bexp = bexp2d[:NB, 0]
    valid = valid2d[:NB, 0]

    tos, w_sorted1d = _sc_scatter(slot, wtok[:, 0])
    w_sorted = w_sorted1d.reshape(P, 1)
    x_sorted = _sc_gather_x(hidden_states, tos)

    y_sorted = pl.pallas_call(
        _moe_kernel,
        grid_spec=pltpu.PrefetchScalarGridSpec(
            num_scalar_prefetch=2,
            grid=(NB,),
            in_specs=[
                pl.BlockSpec((BT, D), lambda i, bexp, valid: (i, 0)),
                pl.BlockSpec((BT, 1), lambda i, bexp, valid: (i, 0)),
                pl.BlockSpec((1, D, F), lambda i, bexp, valid: (bexp[i], 0, 0)),
                pl.BlockSpec((1, D, F), lambda i, bexp, valid: (bexp[i], 0, 0)),
                pl.BlockSpec((1, F, D), lambda i, bexp, valid: (bexp[i], 0, 0)),
            ],
            out_specs=pl.BlockSpec((BT, D), lambda i, bexp, valid: (i, 0)),
        ),
        out_shape=jax.ShapeDtypeStruct((P, D), jnp.float32),
    )(bexp, valid, x_sorted, w_sorted, w1, w3, w2)

    routed = _sc_gather_y(y_sorted, slot)

    BS = 256
    return pl.pallas_call(
        _shared_kernel,
        grid=(T // BS,),
        in_specs=[
            pl.BlockSpec((BS, D), lambda i: (i, 0)),
            pl.BlockSpec((BS, D), lambda i: (i, 0)),
            pl.BlockSpec((D, F), lambda i: (0, 0)),
            pl.BlockSpec((D, F), lambda i: (0, 0)),
            pl.BlockSpec((F, D), lambda i: (0, 0)),
        ],
        out_specs=pl.BlockSpec((BS, D), lambda i: (i, 0)),
        out_shape=jax.ShapeDtypeStruct((T, D), jnp.float32),
    )(hidden_states, routed, shared_w1, shared_w3, shared_w2)


# speedup vs baseline: 1.8788x; 1.0021x over previous
import functools

import jax
import jax.numpy as jnp
from jax import lax
from jax.experimental import pallas as pl
from jax.experimental.pallas import tpu as pltpu
from jax.experimental.pallas import tpu_sc as plsc

T, D, F, E = 2048, 1024, 2048, 8
BT = 256
NB = 16
P = NB * BT
RC = 128
TC = T // RC
NBP = 32


def _route_kernel(logits_ref, slot_ref, w_ref, bexp_ref, valid_ref):
    logits = logits_ref[...]
    lane = lax.broadcasted_iota(jnp.int32, (T, E), 1)
    m = jnp.max(logits, axis=1, keepdims=True)
    e_idx = jnp.min(jnp.where(logits == m, lane, E), axis=1, keepdims=True)
    onehot = (lane == e_idx).astype(jnp.float32)
    counts = jnp.sum(onehot, axis=0, keepdims=True)
    nblk = jnp.floor((counts + (BT - 1)) / BT)
    ii = lax.broadcasted_iota(jnp.int32, (E, E), 0)
    jj = lax.broadcasted_iota(jnp.int32, (E, E), 1)
    excl = (ii < jj).astype(jnp.float32)
    bstart = jnp.dot(nblk, excl, preferred_element_type=jnp.float32)
    rstart = bstart * BT

    ti = lax.broadcasted_iota(jnp.int32, (T, T), 0)
    tj = lax.broadcasted_iota(jnp.int32, (T, T), 1)
    tril = (tj < ti).astype(jnp.float32)
    csum = jnp.dot(tril, onehot, preferred_element_type=jnp.float32)
    rank = jnp.sum(csum * onehot, axis=1, keepdims=True)
    rs_t = jnp.sum(onehot * rstart, axis=1, keepdims=True)
    slot_ref[...] = (rs_t + rank).astype(jnp.int32)
    w_ref[...] = jax.nn.sigmoid(m)

    bi = lax.broadcasted_iota(jnp.int32, (NBP, E), 0).astype(jnp.float32)
    lane2 = lax.broadcasted_iota(jnp.int32, (NBP, E), 1)
    ind = (bi >= bstart) & (bi < bstart + nblk)
    lastexp = jnp.max(jnp.where(counts > 0.0, lane2[:1], 0), axis=1, keepdims=True)
    used = jnp.sum(ind.astype(jnp.int32), axis=1, keepdims=True)
    bexp = jnp.sum(jnp.where(ind, lane2, 0), axis=1, keepdims=True)
    bexp_ref[...] = jnp.where(used > 0, bexp, lastexp)
    valid_ref[...] = used


def _moe_kernel(bexp_ref, valid_ref, xs_ref, ws_ref, w1_ref, w3_ref, w2_ref, y_ref):
    del bexp_ref
    i = pl.program_id(0)

    @pl.when(valid_ref[i] > 0)
    def _():
        x = (xs_ref[...] * ws_ref[...]).astype(jnp.bfloat16)
        g = jnp.dot(x, w1_ref[0].astype(jnp.bfloat16), preferred_element_type=jnp.float32)
        u = jnp.dot(x, w3_ref[0].astype(jnp.bfloat16), preferred_element_type=jnp.float32)
        h = ((g * jax.nn.sigmoid(g)) * u).astype(jnp.bfloat16)
        y_ref[...] = jnp.dot(h, w2_ref[0].astype(jnp.bfloat16), preferred_element_type=jnp.float32)


def _shared_kernel(x_ref, routed_ref, w1_ref, w3_ref, w2_ref, y_ref):
    x = x_ref[...].astype(jnp.bfloat16)
    w1b = w1_ref[...].astype(jnp.bfloat16)
    w3b = w3_ref[...].astype(jnp.bfloat16)
    g = jnp.dot(x, w1b, preferred_element_type=jnp.float32)
    u = jnp.dot(x, w3b, preferred_element_type=jnp.float32)
    h = ((g * jax.nn.sigmoid(g)) * u).astype(jnp.bfloat16)
    w2b = w2_ref[...].astype(jnp.bfloat16)
    y_ref[...] = routed_ref[...] + jnp.dot(h, w2b, preferred_element_type=jnp.float32)


_SC_MESH = plsc.VectorSubcoreMesh(core_axis_name="c", subcore_axis_name="s")
_SC_PARAMS = pltpu.CompilerParams(needs_layout_passes=False)
NW = 32
GCH = 32


def _wid():
    return lax.axis_index("s") * 2 + lax.axis_index("c")


@functools.partial(
    pl.kernel,
    out_type=[
        jax.ShapeDtypeStruct((P,), jnp.int32),
        jax.ShapeDtypeStruct((P,), jnp.float32),
    ],
    mesh=_SC_MESH,
    scratch_types=[
        pltpu.VMEM((T,), jnp.int32),
        pltpu.VMEM((T,), jnp.float32),
        pltpu.VMEM((P,), jnp.int32),
        pltpu.VMEM((P,), jnp.float32),
    ],
    compiler_params=_SC_PARAMS,
)
def _sc_scatter(slot_hbm, w_hbm, tos_hbm, wsort_hbm, slot_v, w_v, tos_v, wsort_v):
    @pl.when(_wid() == 0)
    def _():
        pltpu.sync_copy(slot_hbm, slot_v)
        pltpu.sync_copy(w_hbm, w_v)

        @pl.loop(0, P // 16)
        def _(i):
            tos_v[pl.ds(i * 16, 16)] = (i * 16 + lax.iota(jnp.int32, 16)) & (T - 1)
            wsort_v[pl.ds(i * 16, 16)] = jnp.zeros((16,), jnp.float32)

        @pl.loop(0, T // 16)
        def _(i):
            s = slot_v[pl.ds(i * 16, 16)]
            t = i * 16 + lax.iota(jnp.int32, 16)
            plsc.store_scatter(tos_v, [s], t)
            plsc.store_scatter(wsort_v, [s], w_v[pl.ds(i * 16, 16)])

        pltpu.sync_copy(tos_v, tos_hbm)
        pltpu.sync_copy(wsort_v, wsort_hbm)


def _make_row_gather(nrows):
    gw = nrows // NW
    gnc = gw // GCH

    @functools.partial(
        pl.kernel,
        out_type=jax.ShapeDtypeStruct((nrows, D), jnp.float32),
        mesh=_SC_MESH,
        scratch_types=[
            pltpu.VMEM((gw,), jnp.int32),
            pltpu.VMEM((GCH, D), jnp.float32),
            pltpu.VMEM((GCH, D), jnp.float32),
            pltpu.SemaphoreType.DMA,
            pltpu.SemaphoreType.DMA,
        ],
    )
    def gather(src_hbm, idx_hbm, out_hbm, idx_v, b0, b1, semg, semo):
        base = _wid() * gw
        pltpu.sync_copy(idx_hbm.at[pl.ds(base, gw)], idx_v)
        buf = (b0, b1)

        def start(k):
            return pltpu.async_copy(
                src_hbm.at[idx_v.at[pl.ds(k * GCH, GCH)]], buf[k % 2], semg)

        g = [None] * gnc
        o = [None] * gnc
        g[0] = start(0)
        if gnc > 1:
            g[1] = start(1)
        for k in range(gnc):
            g[k].wait()
            o[k] = pltpu.async_copy(
                buf[k % 2], out_hbm.at[pl.ds(base + k * GCH, GCH)], semo)
            if k + 2 < gnc:
                o[k].wait()
                g[k + 2] = start(k + 2)
        for k in range(max(0, gnc - 2), gnc):
            o[k].wait()

    return gather


_sc_gather_x = _make_row_gather(P)
_sc_gather_y = _make_row_gather(T)


def kernel(hidden_states, router_w, w1, w3, w2, shared_w1, shared_w3, shared_w2):
    logits = hidden_states @ router_w

    slot2d, wtok, bexp2d, valid2d = pl.pallas_call(
        _route_kernel,
        out_shape=[
            jax.ShapeDtypeStruct((T, 1), jnp.int32),
            jax.ShapeDtypeStruct((T, 1), jnp.float32),
            jax.ShapeDtypeStruct((NBP, 1), jnp.int32),
            jax.ShapeDtypeStruct((NBP, 1), jnp.int32),
        ],
    )(logits)
    slot = slot2d[:, 0]
    bexp = bexp2d[:NB, 0]
    valid = valid2d[:NB, 0]

    tos, w_sorted1d = _sc_scatter(slot, wtok[:, 0])
    w_sorted = w_sorted1d.reshape(P, 1)
    x_sorted = _sc_gather_x(hidden_states, tos)

    y_sorted = pl.pallas_call(
        _moe_kernel,
        grid_spec=pltpu.PrefetchScalarGridSpec(
            num_scalar_prefetch=2,
            grid=(NB,),
            in_specs=[
                pl.BlockSpec((BT, D), lambda i, bexp, valid: (i, 0)),
                pl.BlockSpec((BT, 1), lambda i, bexp, valid: (i, 0)),
                pl.BlockSpec((1, D, F), lambda i, bexp, valid: (bexp[i], 0, 0)),
                pl.BlockSpec((1, D, F), lambda i, bexp, valid: (bexp[i], 0, 0)),
                pl.BlockSpec((1, F, D), lambda i, bexp, valid: (bexp[i], 0, 0)),
            ],
            out_specs=pl.BlockSpec((BT, D), lambda i, bexp, valid: (i, 0)),
        ),
        out_shape=jax.ShapeDtypeStruct((P, D), jnp.float32),
    )(bexp, valid, x_sorted, w_sorted, w1, w3, w2)

    routed = _sc_gather_y(y_sorted, slot)

    BS = 512
    return pl.pallas_call(
        _shared_kernel,
        grid=(T // BS,),
        in_specs=[
            pl.BlockSpec((BS, D), lambda i: (i, 0)),
            pl.BlockSpec((BS, D), lambda i: (i, 0)),
            pl.BlockSpec((D, F), lambda i: (0, 0)),
            pl.BlockSpec((D, F), lambda i: (0, 0)),
            pl.BlockSpec((F, D), lambda i: (0, 0)),
        ],
        out_specs=pl.BlockSpec((BS, D), lambda i: (i, 0)),
        out_shape=jax.ShapeDtypeStruct((T, D), jnp.float32),
    )(hidden_states, routed, shared_w1, shared_w3, shared_w2)


# speedup vs baseline: 1.8822x; 1.0018x over previous
import functools

import jax
import jax.numpy as jnp
from jax import lax
from jax.experimental import pallas as pl
from jax.experimental.pallas import tpu as pltpu
from jax.experimental.pallas import tpu_sc as plsc

T, D, F, E = 2048, 1024, 2048, 8
BT = 256
NB = 16
P = NB * BT
RC = 128
TC = T // RC
NBP = 32


def _route_kernel(logits_ref, slot_ref, w_ref, bexp_ref, valid_ref):
    logits = logits_ref[...]
    lane = lax.broadcasted_iota(jnp.int32, (T, E), 1)
    m = jnp.max(logits, axis=1, keepdims=True)
    e_idx = jnp.min(jnp.where(logits == m, lane, E), axis=1, keepdims=True)
    onehot = (lane == e_idx).astype(jnp.float32)
    counts = jnp.sum(onehot, axis=0, keepdims=True)
    nblk = jnp.floor((counts + (BT - 1)) / BT)
    ii = lax.broadcasted_iota(jnp.int32, (E, E), 0)
    jj = lax.broadcasted_iota(jnp.int32, (E, E), 1)
    excl = (ii < jj).astype(jnp.float32)
    bstart = jnp.dot(nblk, excl, preferred_element_type=jnp.float32)
    rstart = bstart * BT

    ti = lax.broadcasted_iota(jnp.int32, (T, T), 0)
    tj = lax.broadcasted_iota(jnp.int32, (T, T), 1)
    tril = (tj < ti).astype(jnp.float32)
    csum = jnp.dot(tril, onehot, preferred_element_type=jnp.float32)
    rank = jnp.sum(csum * onehot, axis=1, keepdims=True)
    rs_t = jnp.sum(onehot * rstart, axis=1, keepdims=True)
    slot_ref[...] = (rs_t + rank).astype(jnp.int32)
    w_ref[...] = jax.nn.sigmoid(m)

    bi = lax.broadcasted_iota(jnp.int32, (NBP, E), 0).astype(jnp.float32)
    lane2 = lax.broadcasted_iota(jnp.int32, (NBP, E), 1)
    ind = (bi >= bstart) & (bi < bstart + nblk)
    lastexp = jnp.max(jnp.where(counts > 0.0, lane2[:1], 0), axis=1, keepdims=True)
    used = jnp.sum(ind.astype(jnp.int32), axis=1, keepdims=True)
    bexp = jnp.sum(jnp.where(ind, lane2, 0), axis=1, keepdims=True)
    bexp_ref[...] = jnp.where(used > 0, bexp, lastexp)
    valid_ref[...] = used


def _moe_kernel(bexp_ref, valid_ref, xs_ref, ws_ref, w1_ref, w3_ref, w2_ref, y_ref):
    del bexp_ref
    i = pl.program_id(0)

    @pl.when(valid_ref[i] > 0)
    def _():
        x = (xs_ref[...] * ws_ref[...]).astype(jnp.bfloat16)
        g = jnp.dot(x, w1_ref[0].astype(jnp.bfloat16), preferred_element_type=jnp.float32)
        u = jnp.dot(x, w3_ref[0].astype(jnp.bfloat16), preferred_element_type=jnp.float32)
        h = ((g * jax.nn.sigmoid(g)) * u).astype(jnp.bfloat16)
        y_ref[...] = jnp.dot(h, w2_ref[0].astype(jnp.bfloat16), preferred_element_type=jnp.float32)


def _shared_kernel(x_ref, routed_ref, w1_ref, w3_ref, w2_ref, y_ref):
    x = x_ref[...].astype(jnp.bfloat16)
    w1b = w1_ref[...].astype(jnp.bfloat16)
    w3b = w3_ref[...].astype(jnp.bfloat16)
    g = jnp.dot(x, w1b, preferred_element_type=jnp.float32)
    u = jnp.dot(x, w3b, preferred_element_type=jnp.float32)
    h = ((g * jax.nn.sigmoid(g)) * u).astype(jnp.bfloat16)
    w2b = w2_ref[...].astype(jnp.bfloat16)
    y_ref[...] = routed_ref[...] + jnp.dot(h, w2b, preferred_element_type=jnp.float32)


_SC_MESH = plsc.VectorSubcoreMesh(core_axis_name="c", subcore_axis_name="s")
_SC_PARAMS = pltpu.CompilerParams(needs_layout_passes=False)
NW = 32
GCH = 32


def _wid():
    return lax.axis_index("s") * 2 + lax.axis_index("c")


GWX = P // NW
GNCX = GWX // GCH


@functools.partial(
    pl.kernel,
    out_type=[
        jax.ShapeDtypeStruct((P, D), jnp.float32),
        jax.ShapeDtypeStruct((P,), jnp.float32),
    ],
    mesh=_SC_MESH,
    scratch_types=[
        pltpu.VMEM((T,), jnp.int32),
        pltpu.VMEM((T,), jnp.float32),
        pltpu.VMEM((GWX,), jnp.int32),
        pltpu.VMEM((GWX,), jnp.float32),
        pltpu.VMEM((GCH, D), jnp.float32),
        pltpu.VMEM((GCH, D), jnp.float32),
        pltpu.SemaphoreType.DMA,
        pltpu.SemaphoreType.DMA,
    ],
    compiler_params=_SC_PARAMS,
)
def _sc_dispatch(x_hbm, slot_hbm, w_hbm, out_hbm, wsort_hbm,
                 slot_v, w_v, idx_v, ws_v, b0, b1, semg, semo):
    base = _wid() * GWX
    pltpu.sync_copy(slot_hbm, slot_v)
    pltpu.sync_copy(w_hbm, w_v)

    @pl.loop(0, GWX // 16)
    def _(i):
        idx_v[pl.ds(i * 16, 16)] = (base + i * 16 + lax.iota(jnp.int32, 16)) & (T - 1)
        ws_v[pl.ds(i * 16, 16)] = jnp.zeros((16,), jnp.float32)

    @pl.loop(0, T // 16)
    def _(i):
        s = slot_v[pl.ds(i * 16, 16)]
        msk = (s >= base) & (s < base + GWX)
        sl = s - base
        t = i * 16 + lax.iota(jnp.int32, 16)
        plsc.store_scatter(idx_v, [sl], t, mask=msk)
        plsc.store_scatter(ws_v, [sl], w_v[pl.ds(i * 16, 16)], mask=msk)

    pltpu.sync_copy(ws_v, wsort_hbm.at[pl.ds(base, GWX)])
    buf = (b0, b1)

    def start(k):
        return pltpu.async_copy(
            x_hbm.at[idx_v.at[pl.ds(k * GCH, GCH)]], buf[k % 2], semg)

    g = [None] * GNCX
    o = [None] * GNCX
    g[0] = start(0)
    g[1] = start(1)
    for k in range(GNCX):
        g[k].wait()
        o[k] = pltpu.async_copy(
            buf[k % 2], out_hbm.at[pl.ds(base + k * GCH, GCH)], semo)
        if k + 2 < GNCX:
            o[k].wait()
            g[k + 2] = start(k + 2)
    for k in range(max(0, GNCX - 2), GNCX):
        o[k].wait()


def _make_row_gather(nrows):
    gw = nrows // NW
    gnc = gw // GCH

    @functools.partial(
        pl.kernel,
        out_type=jax.ShapeDtypeStruct((nrows, D), jnp.float32),
        mesh=_SC_MESH,
        scratch_types=[
            pltpu.VMEM((gw,), jnp.int32),
            pltpu.VMEM((GCH, D), jnp.float32),
            pltpu.VMEM((GCH, D), jnp.float32),
            pltpu.SemaphoreType.DMA,
            pltpu.SemaphoreType.DMA,
        ],
    )
    def gather(src_hbm, idx_hbm, out_hbm, idx_v, b0, b1, semg, semo):
        base = _wid() * gw
        pltpu.sync_copy(idx_hbm.at[pl.ds(base, gw)], idx_v)
        buf = (b0, b1)

        def start(k):
            return pltpu.async_copy(
                src_hbm.at[idx_v.at[pl.ds(k * GCH, GCH)]], buf[k % 2], semg)

        g = [None] * gnc
        o = [None] * gnc
        g[0] = start(0)
        if gnc > 1:
            g[1] = start(1)
        for k in range(gnc):
            g[k].wait()
            o[k] = pltpu.async_copy(
                buf[k % 2], out_hbm.at[pl.ds(base + k * GCH, GCH)], semo)
            if k + 2 < gnc:
                o[k].wait()
                g[k + 2] = start(k + 2)
        for k in range(max(0, gnc - 2), gnc):
            o[k].wait()

    return gather


_sc_gather_y = _make_row_gather(T)


def kernel(hidden_states, router_w, w1, w3, w2, shared_w1, shared_w3, shared_w2):
    logits = hidden_states @ router_w

    slot2d, wtok, bexp2d, valid2d = pl.pallas_call(
        _route_kernel,
        out_shape=[
            jax.ShapeDtypeStruct((T, 1), jnp.int32),
            jax.ShapeDtypeStruct((T, 1), jnp.float32),
            jax.ShapeDtypeStruct((NBP, 1), jnp.int32),
            jax.ShapeDtypeStruct((NBP, 1), jnp.int32),
        ],
    )(logits)
    slot = slot2d[:, 0]
    bexp = bexp2d[:NB, 0]
    valid = valid2d[:NB, 0]

    x_sorted, w_sorted1d = _sc_dispatch(hidden_states, slot, wtok[:, 0])
    w_sorted = w_sorted1d.reshape(P, 1)

    y_sorted = pl.pallas_call(
        _moe_kernel,
        grid_spec=pltpu.PrefetchScalarGridSpec(
            num_scalar_prefetch=2,
            grid=(NB,),
            in_specs=[
                pl.BlockSpec((BT, D), lambda i, bexp, valid: (i, 0)),
                pl.BlockSpec((BT, 1), lambda i, bexp, valid: (i, 0)),
                pl.BlockSpec((1, D, F), lambda i, bexp, valid: (bexp[i], 0, 0)),
                pl.BlockSpec((1, D, F), lambda i, bexp, valid: (bexp[i], 0, 0)),
                pl.BlockSpec((1, F, D), lambda i, bexp, valid: (bexp[i], 0, 0)),
            ],
            out_specs=pl.BlockSpec((BT, D), lambda i, bexp, valid: (i, 0)),
        ),
        out_shape=jax.ShapeDtypeStruct((P, D), jnp.float32),
    )(bexp, valid, x_sorted, w_sorted, w1, w3, w2)

    routed = _sc_gather_y(y_sorted, slot)

    BS = 512
    return pl.pallas_call(
        _shared_kernel,
        grid=(T // BS,),
        in_specs=[
            pl.BlockSpec((BS, D), lambda i: (i, 0)),
            pl.BlockSpec((BS, D), lambda i: (i, 0)),
            pl.BlockSpec((D, F), lambda i: (0, 0)),
            pl.BlockSpec((D, F), lambda i: (0, 0)),
            pl.BlockSpec((F, D), lambda i: (0, 0)),
        ],
        out_specs=pl.BlockSpec((BS, D), lambda i: (i, 0)),
        out_shape=jax.ShapeDtypeStruct((T, D), jnp.float32),
    )(hidden_states, routed, shared_w1, shared_w3, shared_w2)
